# baseline (speedup 1.0000x reference)
import functools

import jax
import jax.numpy as jnp
from jax import lax
from jax.experimental import pallas as pl
from jax.experimental.pallas import tpu as pltpu
from jax.experimental.pallas import tpu_sc as plsc

N = 10000
NP = 10240
E = 320000
F = 128
D = 16
G = 64
NC = 2
NS = 16
NW = NC * NS
CH = 128
EPT = 10240
NCH = EPT // CH
EPAD = NW * EPT
STRIPE = NP // NS


def _mm1_body(x_ref, w_ref, o_ref):
    h = jnp.dot(x_ref[...], w_ref[...], preferred_element_type=jnp.float32)
    o_ref[0:N, :] = h
    o_ref[N:NP, :] = jnp.zeros((NP - N, D), jnp.float32)


_mm1 = pl.pallas_call(
    _mm1_body,
    out_shape=jax.ShapeDtypeStruct((NP, D), jnp.float32),
)


def _mm2_body(p_ref, w_ref, o_ref):
    a = jax.nn.relu(p_ref[0:NP, :] + p_ref[NP:2 * NP, :])
    o_ref[...] = jnp.dot(a, w_ref[...], preferred_element_type=jnp.float32)


_mm2 = pl.pallas_call(
    _mm2_body,
    out_shape=jax.ShapeDtypeStruct((NP, D), jnp.float32),
)


PG = 128


def _final_body(p_ref, wfc_ref, o_ref):
    sums = p_ref[0:G, :] + p_ref[2 * PG:2 * PG + G, :]
    cnts = p_ref[PG:PG + G, :]
    pooled = sums / jnp.maximum(cnts, 1.0)
    o_ref[...] = jax.nn.sigmoid(
        jnp.dot(pooled, wfc_ref[...], preferred_element_type=jnp.float32))


_final = pl.pallas_call(
    _final_body,
    out_shape=jax.ShapeDtypeStruct((G, 1), jnp.float32),
)


NBUF = 8
LAG = 4


def _edge_loop(h_hbm, src_v, dst_v, bufs, acc, gsems, ssems):
    gd = [None] * NBUF
    sd = [None] * NBUF
    for t in range(NCH + LAG):
        if t < NCH:
            b = t % NBUF
            if t >= NBUF:
                sd[b].wait()
            gd[b] = pltpu.async_copy(h_hbm.at[src_v.at[t]], bufs[b],
                                     gsems.at[b])
        u = t - LAG
        if u >= 0:
            bu = u % NBUF
            gd[bu].wait()
            sd[bu] = pltpu.async_copy(bufs[bu], acc.at[dst_v.at[u]],
                                      ssems.at[bu], add=True)
    for b in range(NBUF):
        sd[b].wait()


def _zero_fill(zero_v, acc, s):
    for i in range(CH):
        zero_v[i, :] = jnp.zeros((D,), jnp.float32)
    for k in range(STRIPE // CH):
        pltpu.sync_copy(zero_v, acc.at[pl.ds(s * STRIPE + k * CH, CH)])


def _scatter_body(h_hbm, src_hbm, dst_hbm, out_hbm,
                  src_v, dst_v, bufs, zero_v, acc, gsems, ssems):
    c = lax.axis_index("c")
    s = lax.axis_index("s")
    wid = s * NC + c

    _zero_fill(zero_v, acc, s)
    pltpu.sync_copy(src_hbm.at[wid], src_v)
    pltpu.sync_copy(dst_hbm.at[wid], dst_v)
    plsc.subcore_barrier()

    _edge_loop(h_hbm, src_v, dst_v, bufs, acc, gsems, ssems)

    plsc.subcore_barrier()
    pltpu.sync_copy(acc.at[pl.ds(s * STRIPE, STRIPE)],
                    out_hbm.at[pl.ds(c * NP + s * STRIPE, STRIPE)])


def _scatter_pool_body(h_hbm, src_hbm, dst_hbm, bat_hbm, out_hbm,
                       src_v, dst_v, bat_v, bufs, zero_v, ones_v,
                       acc, pool_v, pool_c, gsems, ssems):
    c = lax.axis_index("c")
    s = lax.axis_index("s")
    wid = s * NC + c

    _zero_fill(zero_v, acc, s)
    for i in range(CH):
        ones_v[i, :] = jnp.ones((D,), jnp.float32)

    @pl.when(s == 0)
    def _():
        pltpu.sync_copy(zero_v, pool_v)
        pltpu.sync_copy(zero_v, pool_c)

    pltpu.sync_copy(src_hbm.at[wid], src_v)
    pltpu.sync_copy(dst_hbm.at[wid], dst_v)
    pltpu.sync_copy(bat_hbm.at[pl.ds(s * (STRIPE // CH), STRIPE // CH)], bat_v)
    plsc.subcore_barrier()

    _edge_loop(h_hbm, src_v, dst_v, bufs, acc, gsems, ssems)

    plsc.subcore_barrier()
    KP = STRIPE // CH
    cps = [pltpu.async_copy(acc.at[pl.ds(s * STRIPE + k * CH, CH)],
                            bufs[k], gsems.at[k]) for k in range(KP)]
    pend = []
    for k in range(KP):
        cps[k].wait()
        pend.append(pltpu.async_copy(bufs[k], pool_v.at[bat_v.at[k]],
                                     ssems.at[k], add=True))
        pend.append(pltpu.async_copy(ones_v, pool_c.at[bat_v.at[k]],
                                     gsems.at[k], add=True))
    for d in pend:
        d.wait()
    plsc.subcore_barrier()

    @pl.when(s == 0)
    def _():
        pltpu.sync_copy(pool_v, out_hbm.at[pl.ds(c * 2 * PG, PG)])
        pltpu.sync_copy(pool_c, out_hbm.at[pl.ds(c * 2 * PG + PG, PG)])


@functools.cache
def _scatter():
    return functools.partial(
        pl.kernel,
        out_type=jax.ShapeDtypeStruct((NC * NP, D), jnp.float32),
        mesh=plsc.VectorSubcoreMesh(core_axis_name="c", subcore_axis_name="s",
                                    num_cores=NC, num_subcores=NS),
        scratch_types=[
            pltpu.VMEM((NCH, CH), jnp.int32),
            pltpu.VMEM((NCH, CH), jnp.int32),
            [pltpu.VMEM((CH, D), jnp.float32) for _ in range(NBUF)],
            pltpu.VMEM((CH, D), jnp.float32),
            pltpu.VMEM_SHARED((NP, D), jnp.float32),
            pltpu.SemaphoreType.DMA((NBUF,)),
            pltpu.SemaphoreType.DMA((NBUF,)),
        ],
        compiler_params=pltpu.CompilerParams(use_tc_tiling_on_sc=False),
    )(_scatter_body)


@functools.cache
def _scatter_pool():
    return functools.partial(
        pl.kernel,
        out_type=jax.ShapeDtypeStruct((NC * 2 * PG, D), jnp.float32),
        mesh=plsc.VectorSubcoreMesh(core_axis_name="c", subcore_axis_name="s",
                                    num_cores=NC, num_subcores=NS),
        scratch_types=[
            pltpu.VMEM((NCH, CH), jnp.int32),
            pltpu.VMEM((NCH, CH), jnp.int32),
            pltpu.VMEM((STRIPE // CH, CH), jnp.int32),
            [pltpu.VMEM((CH, D), jnp.float32) for _ in range(NBUF)],
            pltpu.VMEM((CH, D), jnp.float32),
            pltpu.VMEM((CH, D), jnp.float32),
            pltpu.VMEM_SHARED((NP, D), jnp.float32),
            pltpu.VMEM_SHARED((PG, D), jnp.float32),
            pltpu.VMEM_SHARED((PG, D), jnp.float32),
            pltpu.SemaphoreType.DMA((NBUF,)),
            pltpu.SemaphoreType.DMA((NBUF,)),
        ],
        compiler_params=pltpu.CompilerParams(use_tc_tiling_on_sc=False),
    )(_scatter_pool_body)


def kernel(x, edge_index, batch, W1, W2, Wfc):
    pad = EPAD - E
    src_p = jnp.pad(edge_index[0], (0, pad), constant_values=N).reshape(NW, NCH, CH)
    dst_p = jnp.pad(edge_index[1], (0, pad), constant_values=N).reshape(NW, NCH, CH)
    bat_p = jnp.pad(batch, (0, NP - N), constant_values=G).reshape(NP // CH, CH)

    h1 = _mm1(x, W1)
    p1 = _scatter()(h1, src_p, dst_p)
    h2 = _mm2(p1, W2)
    pooled = _scatter_pool()(h2, src_p, dst_p, bat_p)
    return _final(pooled, Wfc)


# speedup vs baseline: 1.8516x; 1.8516x over previous
import functools

import jax
import jax.numpy as jnp
from jax import lax
from jax.experimental import pallas as pl
from jax.experimental.pallas import tpu as pltpu
from jax.experimental.pallas import tpu_sc as plsc

N = 10000
NP = 10240
E = 320000
F = 128
D = 16
G = 64
NC = 2
NS = 16
NW = NC * NS
CH = 128
EPT = 10240
NCH = EPT // CH
EPAD = NW * EPT
STRIPE = NP // NS


def _mm1_body(x_ref, w_ref, o_ref):
    h = jnp.dot(x_ref[...], w_ref[...], preferred_element_type=jnp.float32)
    o_ref[0:N, :] = h
    o_ref[N:NP, :] = jnp.zeros((NP - N, D), jnp.float32)


_mm1 = pl.pallas_call(
    _mm1_body,
    out_shape=jax.ShapeDtypeStruct((NP, D), jnp.float32),
)


def _mm2_body(p_ref, w_ref, o_ref):
    a = jax.nn.relu(p_ref[0:NP, :] + p_ref[NP:2 * NP, :])
    o_ref[...] = jnp.dot(a, w_ref[...], preferred_element_type=jnp.float32)


_mm2 = pl.pallas_call(
    _mm2_body,
    out_shape=jax.ShapeDtypeStruct((NP, D), jnp.float32),
)


PG = 128


def _final_body(p_ref, wfc_ref, o_ref):
    sums = p_ref[0:G, :] + p_ref[2 * PG:2 * PG + G, :]
    cnts = p_ref[PG:PG + G, :]
    pooled = sums / jnp.maximum(cnts, 1.0)
    o_ref[...] = jax.nn.sigmoid(
        jnp.dot(pooled, wfc_ref[...], preferred_element_type=jnp.float32))


_final = pl.pallas_call(
    _final_body,
    out_shape=jax.ShapeDtypeStruct((G, 1), jnp.float32),
)


NBUF = 8
LAG = 4


def _edge_loop(h_hbm, src_v, dst_v, bufs, acc, gsems, ssems):
    gd = [None] * NBUF
    sd = [None] * NBUF
    for t in range(NCH + LAG):
        if t < NCH:
            b = t % NBUF
            if t >= NBUF:
                sd[b].wait()
            gd[b] = pltpu.async_copy(h_hbm.at[src_v.at[t]], bufs[b],
                                     gsems.at[b])
        u = t - LAG
        if u >= 0:
            bu = u % NBUF
            gd[bu].wait()
            sd[bu] = pltpu.async_copy(bufs[bu], acc.at[dst_v.at[u]],
                                      ssems.at[bu], add=True)
    for b in range(NBUF):
        sd[b].wait()


def _zero_fill(zero_v, acc, s):
    for i in range(CH):
        zero_v[i, :] = jnp.zeros((D,), jnp.float32)
    for k in range(STRIPE // CH):
        pltpu.sync_copy(zero_v, acc.at[pl.ds(s * STRIPE + k * CH, CH)])


def _scatter_body(h_hbm, src_hbm, dst_hbm, out_hbm,
                  src_v, dst_v, bufs, zero_v, acc, gsems, ssems):
    c = lax.axis_index("c")
    s = lax.axis_index("s")
    wid = s * NC + c

    _zero_fill(zero_v, acc, s)
    pltpu.sync_copy(src_hbm.at[wid], src_v)
    pltpu.sync_copy(dst_hbm.at[wid], dst_v)
    plsc.subcore_barrier()

    _edge_loop(h_hbm, src_v, dst_v, bufs, acc, gsems, ssems)

    plsc.subcore_barrier()
    pltpu.sync_copy(acc.at[pl.ds(s * STRIPE, STRIPE)],
                    out_hbm.at[pl.ds(c * NP + s * STRIPE, STRIPE)])


def _scatter_pool_body(h_hbm, src_hbm, dst_hbm, bat_hbm, out_hbm,
                       src_v, dst_v, bat_v, bufs, zero_v, ones_v,
                       acc, pool_v, pool_c, gsems, ssems):
    c = lax.axis_index("c")
    s = lax.axis_index("s")
    wid = s * NC + c

    _zero_fill(zero_v, acc, s)
    for i in range(CH):
        ones_v[i, :] = jnp.ones((D,), jnp.float32)

    @pl.when(s == 0)
    def _():
        pltpu.sync_copy(zero_v, pool_v)
        pltpu.sync_copy(zero_v, pool_c)

    pltpu.sync_copy(src_hbm.at[wid], src_v)
    pltpu.sync_copy(dst_hbm.at[wid], dst_v)
    pltpu.sync_copy(bat_hbm.at[pl.ds(s * (STRIPE // CH), STRIPE // CH)], bat_v)
    plsc.subcore_barrier()

    _edge_loop(h_hbm, src_v, dst_v, bufs, acc, gsems, ssems)

    plsc.subcore_barrier()
    KP = STRIPE // CH
    cps = [pltpu.async_copy(acc.at[pl.ds(s * STRIPE + k * CH, CH)],
                            bufs[k], gsems.at[k]) for k in range(KP)]
    pend = []
    for k in range(KP):
        cps[k].wait()
        pend.append(pltpu.async_copy(bufs[k], pool_v.at[bat_v.at[k]],
                                     ssems.at[k], add=True))
        pend.append(pltpu.async_copy(ones_v, pool_c.at[bat_v.at[k]],
                                     gsems.at[k], add=True))
    for d in pend:
        d.wait()
    plsc.subcore_barrier()

    @pl.when(s == 0)
    def _():
        pltpu.sync_copy(pool_v, out_hbm.at[pl.ds(c * 2 * PG, PG)])
        pltpu.sync_copy(pool_c, out_hbm.at[pl.ds(c * 2 * PG + PG, PG)])


@functools.cache
def _scatter():
    return functools.partial(
        pl.kernel,
        out_type=jax.ShapeDtypeStruct((NC * NP, D), jnp.float32),
        mesh=plsc.VectorSubcoreMesh(core_axis_name="c", subcore_axis_name="s",
                                    num_cores=NC, num_subcores=NS),
        scratch_types=[
            pltpu.VMEM((NCH, CH), jnp.int32),
            pltpu.VMEM((NCH, CH), jnp.int32),
            [pltpu.VMEM((CH, D), jnp.float32) for _ in range(NBUF)],
            pltpu.VMEM((CH, D), jnp.float32),
            pltpu.VMEM_SHARED((NP, D), jnp.float32),
            pltpu.SemaphoreType.DMA((NBUF,)),
            pltpu.SemaphoreType.DMA((NBUF,)),
        ],
        compiler_params=pltpu.CompilerParams(use_tc_tiling_on_sc=False),
    )(_scatter_body)


@functools.cache
def _scatter_pool():
    return functools.partial(
        pl.kernel,
        out_type=jax.ShapeDtypeStruct((NC * 2 * PG, D), jnp.float32),
        mesh=plsc.VectorSubcoreMesh(core_axis_name="c", subcore_axis_name="s",
                                    num_cores=NC, num_subcores=NS),
        scratch_types=[
            pltpu.VMEM((NCH, CH), jnp.int32),
            pltpu.VMEM((NCH, CH), jnp.int32),
            pltpu.VMEM((STRIPE // CH, CH), jnp.int32),
            [pltpu.VMEM((CH, D), jnp.float32) for _ in range(NBUF)],
            pltpu.VMEM((CH, D), jnp.float32),
            pltpu.VMEM((CH, D), jnp.float32),
            pltpu.VMEM_SHARED((NP, D), jnp.float32),
            pltpu.VMEM_SHARED((PG, D), jnp.float32),
            pltpu.VMEM_SHARED((PG, D), jnp.float32),
            pltpu.SemaphoreType.DMA((NBUF,)),
            pltpu.SemaphoreType.DMA((NBUF,)),
        ],
        compiler_params=pltpu.CompilerParams(use_tc_tiling_on_sc=False),
    )(_scatter_pool_body)


def kernel(x, edge_index, batch, W1, W2, Wfc):
    pad = EPAD - E
    padidx = N + (jnp.arange(pad, dtype=jnp.int32) % (NP - N))
    src_p = jnp.concatenate([edge_index[0], padidx]).reshape(NW, NCH, CH)
    dst_p = jnp.concatenate([edge_index[1], padidx]).reshape(NW, NCH, CH)
    bat_p = jnp.pad(batch, (0, NP - N), constant_values=G).reshape(NP // CH, CH)

    h1 = _mm1(x, W1)
    p1 = _scatter()(h1, src_p, dst_p)
    h2 = _mm2(p1, W2)
    pooled = _scatter_pool()(h2, src_p, dst_p, bat_p)
    return _final(pooled, Wfc)


# speedup vs baseline: 2.2408x; 1.2102x over previous
import functools

import jax
import jax.numpy as jnp
from jax import lax
from jax.experimental import pallas as pl
from jax.experimental.pallas import tpu as pltpu
from jax.experimental.pallas import tpu_sc as plsc

N = 10000
NP = 10240
E = 320000
F = 128
D = 16
G = 64
NC = 2
NS = 16
NW = NC * NS
CH = 128
EPT = 10240
NCH = EPT // CH
EPAD = NW * EPT
STRIPE = NP // NS


NPK = NP // 8
NRK = N // 8


def _mm1_body(x_ref, w_ref, o_ref):
    o_ref[...] = jnp.zeros((NPK, 8 * D), jnp.float32)
    for u in range(8):
        xu = x_ref[:, u, :]
        o_ref[0:NRK, u * D:(u + 1) * D] = jnp.dot(
            xu, w_ref[...], preferred_element_type=jnp.float32)


_mm1 = pl.pallas_call(
    _mm1_body,
    out_shape=jax.ShapeDtypeStruct((NPK, 8 * D), jnp.float32),
)


def _mm2_body(p_ref, w_ref, o_ref):
    a = jax.nn.relu(p_ref[0:NPK, :] + p_ref[NPK:2 * NPK, :])
    w2t = jnp.concatenate([w_ref[...]] * 8, axis=0)
    w2t = jnp.concatenate([w2t] * 8, axis=1)
    bi = lax.broadcasted_iota(jnp.int32, (8 * D, 8 * D), 0)
    bj = lax.broadcasted_iota(jnp.int32, (8 * D, 8 * D), 1)
    w2big = jnp.where((bi // D) == (bj // D), w2t, 0.0)
    o_ref[...] = jnp.dot(a, w2big, preferred_element_type=jnp.float32)


_mm2 = pl.pallas_call(
    _mm2_body,
    out_shape=jax.ShapeDtypeStruct((NPK, 8 * D), jnp.float32),
)


PG = 128


def _final_body(p_ref, wfc_ref, o_ref):
    sums = p_ref[0:G, :] + p_ref[2 * PG:2 * PG + G, :]
    cnts = p_ref[PG:PG + G, :]
    pooled = sums / jnp.maximum(cnts, 1.0)
    o_ref[...] = jax.nn.sigmoid(
        jnp.dot(pooled, wfc_ref[...], preferred_element_type=jnp.float32))


_final = pl.pallas_call(
    _final_body,
    out_shape=jax.ShapeDtypeStruct((G, 1), jnp.float32),
)


NBUF = 8
LAG = 4


def _edge_loop(h_hbm, src_v, dst_v, bufs, acc, gsems, ssems):
    gd = [None] * NBUF
    sd = [None] * NBUF
    for t in range(NCH + LAG):
        if t < NCH:
            b = t % NBUF
            if t >= NBUF:
                sd[b].wait()
            gd[b] = pltpu.async_copy(h_hbm.at[src_v.at[t]], bufs[b],
                                     gsems.at[b])
        u = t - LAG
        if u >= 0:
            bu = u % NBUF
            gd[bu].wait()
            sd[bu] = pltpu.async_copy(bufs[bu], acc.at[dst_v.at[u]],
                                      ssems.at[bu], add=True)
    for b in range(NBUF):
        sd[b].wait()


def _zero_fill(zero_v, acc, s):
    for i in range(CH):
        zero_v[i, :] = jnp.zeros((D,), jnp.float32)
    for k in range(STRIPE // CH):
        pltpu.sync_copy(zero_v, acc.at[pl.ds(s * STRIPE + k * CH, CH)])


def _scatter_body(h_hbm, src_hbm, dst_hbm, out_hbm,
                  src_v, dst_v, bufs, zero_v, acc, gsems, ssems):
    c = lax.axis_index("c")
    s = lax.axis_index("s")
    wid = s * NC + c
    _zero_fill(zero_v, acc, s)
    pltpu.sync_copy(src_hbm.at[wid], src_v)
    pltpu.sync_copy(dst_hbm.at[wid], dst_v)
    plsc.subcore_barrier()

    _edge_loop(h_hbm, src_v, dst_v, bufs, acc, gsems, ssems)

    plsc.subcore_barrier()
    pltpu.sync_copy(acc.at[pl.ds(s * STRIPE, STRIPE)],
                    out_hbm.at[pl.ds(c * NP + s * STRIPE, STRIPE)])


def _scatter_pool_body(h_hbm, src_hbm, dst_hbm, bat_hbm, out_hbm,
                       src_v, dst_v, bat_v, bufs, zero_v, ones_v,
                       acc, pool_v, pool_c, gsems, ssems):
    c = lax.axis_index("c")
    s = lax.axis_index("s")
    wid = s * NC + c

    _zero_fill(zero_v, acc, s)
    for i in range(CH):
        ones_v[i, :] = jnp.ones((D,), jnp.float32)

    @pl.when(s == 0)
    def _():
        pltpu.sync_copy(zero_v, pool_v)
        pltpu.sync_copy(zero_v, pool_c)

    pltpu.sync_copy(src_hbm.at[wid], src_v)
    pltpu.sync_copy(dst_hbm.at[wid], dst_v)
    pltpu.sync_copy(bat_hbm.at[pl.ds(s * (STRIPE // CH), STRIPE // CH)], bat_v)
    plsc.subcore_barrier()

    _edge_loop(h_hbm, src_v, dst_v, bufs, acc, gsems, ssems)

    plsc.subcore_barrier()
    KP = STRIPE // CH
    cps = [pltpu.async_copy(acc.at[pl.ds(s * STRIPE + k * CH, CH)],
                            bufs[k], gsems.at[k]) for k in range(KP)]
    pend = []
    for k in range(KP):
        cps[k].wait()
        pend.append(pltpu.async_copy(bufs[k], pool_v.at[bat_v.at[k]],
                                     ssems.at[k], add=True))
        pend.append(pltpu.async_copy(ones_v, pool_c.at[bat_v.at[k]],
                                     gsems.at[k], add=True))
    for d in pend:
        d.wait()
    plsc.subcore_barrier()

    @pl.when(s == 0)
    def _():
        pltpu.sync_copy(pool_v, out_hbm.at[pl.ds(c * 2 * PG, PG)])
        pltpu.sync_copy(pool_c, out_hbm.at[pl.ds(c * 2 * PG + PG, PG)])


@functools.cache
def _scatter():
    return functools.partial(
        pl.kernel,
        out_type=jax.ShapeDtypeStruct((NC * NP, D), jnp.float32),
        mesh=plsc.VectorSubcoreMesh(core_axis_name="c", subcore_axis_name="s",
                                    num_cores=NC, num_subcores=NS),
        scratch_types=[
            pltpu.VMEM((NCH, CH), jnp.int32),
            pltpu.VMEM((NCH, CH), jnp.int32),
            [pltpu.VMEM((CH, D), jnp.float32) for _ in range(NBUF)],
            pltpu.VMEM((CH, D), jnp.float32),
            pltpu.VMEM_SHARED((NP, D), jnp.float32),
            pltpu.SemaphoreType.DMA((NBUF,)),
            pltpu.SemaphoreType.DMA((NBUF,)),
        ],
        compiler_params=pltpu.CompilerParams(use_tc_tiling_on_sc=False),
    )(_scatter_body)


@functools.cache
def _scatter_pool():
    return functools.partial(
        pl.kernel,
        out_type=jax.ShapeDtypeStruct((NC * 2 * PG, D), jnp.float32),
        mesh=plsc.VectorSubcoreMesh(core_axis_name="c", subcore_axis_name="s",
                                    num_cores=NC, num_subcores=NS),
        scratch_types=[
            pltpu.VMEM((NCH, CH), jnp.int32),
            pltpu.VMEM((NCH, CH), jnp.int32),
            pltpu.VMEM((STRIPE // CH, CH), jnp.int32),
            [pltpu.VMEM((CH, D), jnp.float32) for _ in range(NBUF)],
            pltpu.VMEM((CH, D), jnp.float32),
            pltpu.VMEM((CH, D), jnp.float32),
            pltpu.VMEM_SHARED((NP, D), jnp.float32),
            pltpu.VMEM_SHARED((PG, D), jnp.float32),
            pltpu.VMEM_SHARED((PG, D), jnp.float32),
            pltpu.SemaphoreType.DMA((NBUF,)),
            pltpu.SemaphoreType.DMA((NBUF,)),
        ],
        compiler_params=pltpu.CompilerParams(use_tc_tiling_on_sc=False),
    )(_scatter_pool_body)


def kernel(x, edge_index, batch, W1, W2, Wfc):
    pad = EPAD - E
    padidx = N + (jnp.arange(pad, dtype=jnp.int32) % (NP - N))
    src_p = jnp.concatenate([edge_index[0], padidx]).reshape(NW, NCH, CH)
    dst_p = jnp.concatenate([edge_index[1], padidx]).reshape(NW, NCH, CH)
    bat_p = jnp.pad(batch, (0, NP - N), constant_values=G).reshape(NP // CH, CH)

    h1 = _mm1(x.reshape(NRK, 8, F), W1)
    p1 = _scatter()(h1.reshape(NP, D), src_p, dst_p)
    h2 = _mm2(p1.reshape(2 * NPK, 8 * D), W2)
    pooled = _scatter_pool()(h2.reshape(NP, D), src_p, dst_p, bat_p)
    return _final(pooled, Wfc)


# speedup vs baseline: 2.5917x; 1.1566x over previous
import functools

import jax
import jax.numpy as jnp
from jax import lax
from jax.experimental import pallas as pl
from jax.experimental.pallas import tpu as pltpu
from jax.experimental.pallas import tpu_sc as plsc

N = 10000
NP = 10240
E = 320000
F = 128
D = 16
G = 64
NC = 2
NS = 16
NW = NC * NS
CH = 128
EPT = 10240
NCH = EPT // CH
EPAD = NW * EPT
STRIPE = NP // NS


NPK = NP // 8
NRK = N // 8


ECH = E // CH
EPR = EPAD // CH


def _mm1_body(x_ref, w_ref, e_ref, o_ref, osrc_ref, odst_ref):
    o_ref[...] = jnp.zeros((NPK, 8 * D), jnp.float32)
    for u in range(8):
        xu = x_ref[:, u, :]
        o_ref[0:NRK, u * D:(u + 1) * D] = jnp.dot(
            xu, w_ref[...], preferred_element_type=jnp.float32)
    osrc_ref[0:ECH, :] = jnp.reshape(e_ref[0, :], (ECH, CH))
    odst_ref[0:ECH, :] = jnp.reshape(e_ref[1, :], (ECH, CH))
    bi = lax.broadcasted_iota(jnp.int32, (EPR - ECH, CH), 0)
    bj = lax.broadcasted_iota(jnp.int32, (EPR - ECH, CH), 1)
    pads = N + (bi * CH + bj) % (NP - N)
    osrc_ref[ECH:EPR, :] = pads
    odst_ref[ECH:EPR, :] = pads


_mm1 = pl.pallas_call(
    _mm1_body,
    out_shape=(
        jax.ShapeDtypeStruct((NPK, 8 * D), jnp.float32),
        jax.ShapeDtypeStruct((EPR, CH), jnp.int32),
        jax.ShapeDtypeStruct((EPR, CH), jnp.int32),
    ),
)


def _mm2_body(p_ref, w_ref, o_ref):
    a = jax.nn.relu(p_ref[0:NPK, :] + p_ref[NPK:2 * NPK, :])
    w2t = jnp.concatenate([w_ref[...]] * 8, axis=0)
    w2t = jnp.concatenate([w2t] * 8, axis=1)
    bi = lax.broadcasted_iota(jnp.int32, (8 * D, 8 * D), 0)
    bj = lax.broadcasted_iota(jnp.int32, (8 * D, 8 * D), 1)
    w2big = jnp.where((bi // D) == (bj // D), w2t, 0.0)
    o_ref[...] = jnp.dot(a, w2big, preferred_element_type=jnp.float32)


_mm2 = pl.pallas_call(
    _mm2_body,
    out_shape=jax.ShapeDtypeStruct((NPK, 8 * D), jnp.float32),
)


PG = 128


def _final_body(p_ref, wfc_ref, o_ref):
    sums = p_ref[0:G, :] + p_ref[2 * PG:2 * PG + G, :]
    cnts = p_ref[PG:PG + G, :]
    pooled = sums / jnp.maximum(cnts, 1.0)
    o_ref[...] = jax.nn.sigmoid(
        jnp.dot(pooled, wfc_ref[...], preferred_element_type=jnp.float32))


_final = pl.pallas_call(
    _final_body,
    out_shape=jax.ShapeDtypeStruct((G, 1), jnp.float32),
)


NBUF = 8
LAG = 4


def _edge_loop(h_hbm, src_v, dst_v, bufs, acc, gsems, ssems):
    gd = [None] * NBUF
    sd = [None] * NBUF
    for t in range(NCH + LAG):
        if t < NCH:
            b = t % NBUF
            if t >= NBUF:
                sd[b].wait()
            gd[b] = pltpu.async_copy(h_hbm.at[src_v.at[t]], bufs[b],
                                     gsems.at[b])
        u = t - LAG
        if u >= 0:
            bu = u % NBUF
            gd[bu].wait()
            sd[bu] = pltpu.async_copy(bufs[bu], acc.at[dst_v.at[u]],
                                      ssems.at[bu], add=True)
    for b in range(NBUF):
        sd[b].wait()


def _zero_fill(zero_v, acc, s):
    for i in range(CH):
        zero_v[i, :] = jnp.zeros((D,), jnp.float32)
    for k in range(STRIPE // CH):
        pltpu.sync_copy(zero_v, acc.at[pl.ds(s * STRIPE + k * CH, CH)])


def _scatter_body(h_hbm, src_hbm, dst_hbm, out_hbm,
                  src_v, dst_v, bufs, zero_v, acc, gsems, ssems):
    c = lax.axis_index("c")
    s = lax.axis_index("s")
    wid = s * NC + c
    _zero_fill(zero_v, acc, s)
    pltpu.sync_copy(src_hbm.at[wid], src_v)
    pltpu.sync_copy(dst_hbm.at[wid], dst_v)
    plsc.subcore_barrier()

    _edge_loop(h_hbm, src_v, dst_v, bufs, acc, gsems, ssems)

    plsc.subcore_barrier()
    pltpu.sync_copy(acc.at[pl.ds(s * STRIPE, STRIPE)],
                    out_hbm.at[pl.ds(c * NP + s * STRIPE, STRIPE)])


def _scatter_pool_body(h_hbm, src_hbm, dst_hbm, bat_hbm, out_hbm,
                       src_v, dst_v, bat_v, bufs, zero_v, ones_v,
                       acc, pool_v, pool_c, gsems, ssems):
    c = lax.axis_index("c")
    s = lax.axis_index("s")
    wid = s * NC + c

    _zero_fill(zero_v, acc, s)
    for i in range(CH):
        ones_v[i, :] = jnp.ones((D,), jnp.float32)

    @pl.when(s == 0)
    def _():
        pltpu.sync_copy(zero_v, pool_v)
        pltpu.sync_copy(zero_v, pool_c)

    pltpu.sync_copy(src_hbm.at[wid], src_v)
    pltpu.sync_copy(dst_hbm.at[wid], dst_v)
    pltpu.sync_copy(bat_hbm.at[pl.ds(s * (STRIPE // CH), STRIPE // CH)], bat_v)
    plsc.subcore_barrier()

    _edge_loop(h_hbm, src_v, dst_v, bufs, acc, gsems, ssems)

    plsc.subcore_barrier()
    KP = STRIPE // CH
    cps = [pltpu.async_copy(acc.at[pl.ds(s * STRIPE + k * CH, CH)],
                            bufs[k], gsems.at[k]) for k in range(KP)]
    pend = []
    for k in range(KP):
        cps[k].wait()
        pend.append(pltpu.async_copy(bufs[k], pool_v.at[bat_v.at[k]],
                                     ssems.at[k], add=True))
        pend.append(pltpu.async_copy(ones_v, pool_c.at[bat_v.at[k]],
                                     gsems.at[k], add=True))
    for d in pend:
        d.wait()
    plsc.subcore_barrier()

    @pl.when(s == 0)
    def _():
        pltpu.sync_copy(pool_v, out_hbm.at[pl.ds(c * 2 * PG, PG)])
        pltpu.sync_copy(pool_c, out_hbm.at[pl.ds(c * 2 * PG + PG, PG)])


@functools.cache
def _scatter():
    return functools.partial(
        pl.kernel,
        out_type=jax.ShapeDtypeStruct((NC * NP, D), jnp.float32),
        mesh=plsc.VectorSubcoreMesh(core_axis_name="c", subcore_axis_name="s",
                                    num_cores=NC, num_subcores=NS),
        scratch_types=[
            pltpu.VMEM((NCH, CH), jnp.int32),
            pltpu.VMEM((NCH, CH), jnp.int32),
            [pltpu.VMEM((CH, D), jnp.float32) for _ in range(NBUF)],
            pltpu.VMEM((CH, D), jnp.float32),
            pltpu.VMEM_SHARED((NP, D), jnp.float32),
            pltpu.SemaphoreType.DMA((NBUF,)),
            pltpu.SemaphoreType.DMA((NBUF,)),
        ],
        compiler_params=pltpu.CompilerParams(use_tc_tiling_on_sc=False),
    )(_scatter_body)


@functools.cache
def _scatter_pool():
    return functools.partial(
        pl.kernel,
        out_type=jax.ShapeDtypeStruct((NC * 2 * PG, D), jnp.float32),
        mesh=plsc.VectorSubcoreMesh(core_axis_name="c", subcore_axis_name="s",
                                    num_cores=NC, num_subcores=NS),
        scratch_types=[
            pltpu.VMEM((NCH, CH), jnp.int32),
            pltpu.VMEM((NCH, CH), jnp.int32),
            pltpu.VMEM((STRIPE // CH, CH), jnp.int32),
            [pltpu.VMEM((CH, D), jnp.float32) for _ in range(NBUF)],
            pltpu.VMEM((CH, D), jnp.float32),
            pltpu.VMEM((CH, D), jnp.float32),
            pltpu.VMEM_SHARED((NP, D), jnp.float32),
            pltpu.VMEM_SHARED((PG, D), jnp.float32),
            pltpu.VMEM_SHARED((PG, D), jnp.float32),
            pltpu.SemaphoreType.DMA((NBUF,)),
            pltpu.SemaphoreType.DMA((NBUF,)),
        ],
        compiler_params=pltpu.CompilerParams(use_tc_tiling_on_sc=False),
    )(_scatter_pool_body)


def kernel(x, edge_index, batch, W1, W2, Wfc):
    bat_p = jnp.pad(batch, (0, NP - N), constant_values=G).reshape(NP // CH, CH)

    h1, src_f, dst_f = _mm1(x.reshape(NRK, 8, F), W1, edge_index)
    src_p = src_f.reshape(NW, NCH, CH)
    dst_p = dst_f.reshape(NW, NCH, CH)
    p1 = _scatter()(h1.reshape(NP, D), src_p, dst_p)
    h2 = _mm2(p1.reshape(2 * NPK, 8 * D), W2)
    pooled = _scatter_pool()(h2.reshape(NP, D), src_p, dst_p, bat_p)
    return _final(pooled, Wfc)
